# baseline (speedup 1.0000x reference)
import functools

import jax
import jax.numpy as jnp
from jax import lax
from jax.experimental import pallas as pl
from jax.experimental.pallas import tpu as pltpu
from jax.experimental.pallas import tpu_sc as plsc

D = 128
NC, NS = 2, 16
NW = NC * NS
CHUNK = 64
UNIT = NW * CHUNK


def _sc_gather_piece(uids3, iids3, utab, itab, units):
    kpw = len(units)
    rows = NW * kpw * CHUNK
    mesh = plsc.VectorSubcoreMesh(
        core_axis_name="c", subcore_axis_name="s",
        num_cores=NC, num_subcores=NS)

    @functools.partial(
        pl.kernel,
        out_type=[jax.ShapeDtypeStruct((rows, D), jnp.float32),
                  jax.ShapeDtypeStruct((rows, D), jnp.float32)],
        mesh=mesh,
        scratch_types=(
            [pltpu.VMEM((kpw, CHUNK), jnp.int32) for _ in range(2)]
            + [pltpu.VMEM((CHUNK, D), jnp.float32) for _ in range(2 * kpw)]
            + [pltpu.SemaphoreType.DMA for _ in range(6 * kpw)]
        ),
    )
    def k(uid_hbm, iid_hbm, utab_hbm, itab_hbm, outu_hbm, outi_hbm, *rest):
        outs = (outu_hbm, outi_hbm)
        idx = rest[:2]
        bufs = rest[2:2 + 2 * kpw]
        gsem = rest[2 + 2 * kpw:2 + 4 * kpw]
        wsem = rest[2 + 4 * kpw:2 + 6 * kpw]
        isem = rest[2 + 6 * kpw:]
        wid = lax.axis_index("s") * NC + lax.axis_index("c")
        stages = []
        for j, u in enumerate(units):
            stages.append(pltpu.async_copy(
                uid_hbm.at[u, wid], idx[0].at[j], isem[2 * j]))
            stages.append(pltpu.async_copy(
                iid_hbm.at[u, wid], idx[1].at[j], isem[2 * j + 1]))
        for s in stages:
            s.wait()
        gathers = []
        for p, tab in enumerate((utab_hbm, itab_hbm)):
            for j in range(kpw):
                n = p * kpw + j
                gathers.append(pltpu.async_copy(
                    tab.at[idx[p].at[j]], bufs[n], gsem[n]))
        writes = []
        for p in range(2):
            for j in range(kpw):
                n = p * kpw + j
                gathers[n].wait()
                writes.append(pltpu.async_copy(
                    bufs[n],
                    outs[p].at[pl.ds((j * NW + wid) * CHUNK, CHUNK)],
                    wsem[n]))
        for w in writes:
            w.wait()

    return k(uids3, iids3, utab, itab)


def _tc_project_piece(Xu, Xi, Wu, bu, Wi, bi, row0, B, prev=None):
    rows = Xu.shape[0]
    BM = 2048
    nblk = rows // BM
    blk0 = row0 // BM
    dn = (((1,), (1,)), ((), ()))

    def body(xu_ref, xi_ref, wu_ref, bu_ref, wi_ref, bi_ref, *rest):
        ou_ref, oi_ref = rest[-2], rest[-1]
        ou_ref[...] = lax.dot_general(
            xu_ref[...], wu_ref[...], dn,
            preferred_element_type=jnp.float32) + bu_ref[...]
        oi_ref[...] = lax.dot_general(
            xi_ref[...], wi_ref[...], dn,
            preferred_element_type=jnp.float32) + bi_ref[...]

    in_specs = [
        pl.BlockSpec((BM, D), lambda i: (i, 0)),
        pl.BlockSpec((BM, D), lambda i: (i, 0)),
        pl.BlockSpec((D, D), lambda i: (0, 0)),
        pl.BlockSpec((1, D), lambda i: (0, 0)),
        pl.BlockSpec((D, D), lambda i: (0, 0)),
        pl.BlockSpec((1, D), lambda i: (0, 0)),
    ]
    inputs = [Xu, Xi, Wu, bu.reshape(1, D), Wi, bi.reshape(1, D)]
    aliases = {}
    if prev is not None:
        in_specs += [
            pl.BlockSpec((BM, D), lambda i, b0=blk0: (b0 + i, 0)),
            pl.BlockSpec((BM, D), lambda i, b0=blk0: (b0 + i, 0)),
        ]
        inputs += [prev[0], prev[1]]
        aliases = {6: 0, 7: 1}

    return pl.pallas_call(
        body,
        grid=(nblk,),
        in_specs=in_specs,
        out_specs=[
            pl.BlockSpec((BM, D), lambda i, b0=blk0: (b0 + i, 0)),
            pl.BlockSpec((BM, D), lambda i, b0=blk0: (b0 + i, 0)),
        ],
        out_shape=[
            jax.ShapeDtypeStruct((B, D), jnp.float32),
            jax.ShapeDtypeStruct((B, D), jnp.float32),
        ],
        input_output_aliases=aliases,
    )(*inputs)


def kernel(user_ids, item_ids, user_embedding, item_embedding,
           W_user, b_user, W_item, b_item):
    B = user_ids.shape[0]
    nunits = B // UNIT
    uids3 = user_ids.astype(jnp.int32).reshape(nunits, NW, CHUNK)
    iids3 = item_ids.astype(jnp.int32).reshape(nunits, NW, CHUNK)
    units0 = tuple(range(5))
    units1 = tuple(range(5, nunits))

    X0u, X0i = _sc_gather_piece(uids3, iids3, user_embedding,
                                item_embedding, units0)
    X1u, X1i = _sc_gather_piece(uids3, iids3, user_embedding,
                                item_embedding, units1)
    out0 = _tc_project_piece(X0u, X0i, W_user, b_user, W_item, b_item,
                             0, B)
    ou, oi = _tc_project_piece(X1u, X1i, W_user, b_user, W_item, b_item,
                               len(units0) * UNIT, B, prev=out0)
    return (ou, oi)


# speedup vs baseline: 1.0801x; 1.0801x over previous
import functools

import jax
import jax.numpy as jnp
from jax import lax
from jax.experimental import pallas as pl
from jax.experimental.pallas import tpu as pltpu
from jax.experimental.pallas import tpu_sc as plsc

D = 128
NC, NS = 2, 16
NW = NC * NS
CHUNK = 128
UNIT = NW * CHUNK


def _sc_gather_piece(uids3, iids3, utab, itab, units):
    kpw = len(units)
    rows = NW * kpw * CHUNK
    mesh = plsc.VectorSubcoreMesh(
        core_axis_name="c", subcore_axis_name="s",
        num_cores=NC, num_subcores=NS)

    @functools.partial(
        pl.kernel,
        out_type=jax.ShapeDtypeStruct((2, rows, D), jnp.float32),
        mesh=mesh,
        scratch_types=(
            [pltpu.VMEM((kpw, CHUNK), jnp.int32) for _ in range(2)]
            + [pltpu.VMEM((CHUNK, D), jnp.float32) for _ in range(2 * kpw)]
            + [pltpu.SemaphoreType.DMA for _ in range(6 * kpw)]
        ),
    )
    def k(uid_hbm, iid_hbm, utab_hbm, itab_hbm, out_hbm, *rest):
        idx = rest[:2]
        bufs = rest[2:2 + 2 * kpw]
        gsem = rest[2 + 2 * kpw:2 + 4 * kpw]
        wsem = rest[2 + 4 * kpw:2 + 6 * kpw]
        isem = rest[2 + 6 * kpw:]
        wid = lax.axis_index("s") * NC + lax.axis_index("c")
        stages = []
        for j, u in enumerate(units):
            stages.append(pltpu.async_copy(
                uid_hbm.at[u, wid], idx[0].at[j], isem[2 * j]))
            stages.append(pltpu.async_copy(
                iid_hbm.at[u, wid], idx[1].at[j], isem[2 * j + 1]))
        for s in stages:
            s.wait()
        gathers = []
        for p, tab in enumerate((utab_hbm, itab_hbm)):
            for j in range(kpw):
                n = p * kpw + j
                gathers.append(pltpu.async_copy(
                    tab.at[idx[p].at[j]], bufs[n], gsem[n]))
        writes = []
        for p in range(2):
            for j in range(kpw):
                n = p * kpw + j
                gathers[n].wait()
                writes.append(pltpu.async_copy(
                    bufs[n],
                    out_hbm.at[p, pl.ds((j * NW + wid) * CHUNK, CHUNK)],
                    wsem[n]))
        for w in writes:
            w.wait()

    return k(uids3, iids3, utab, itab)


def _tc_project_piece(X, Wu, bu, Wi, bi, row0, B, prev=None):
    rows = X.shape[1]
    BM = 2048
    nblk = rows // BM
    blk0 = row0 // BM
    dn = (((1,), (1,)), ((), ()))

    def body(x_ref, wu_ref, bu_ref, wi_ref, bi_ref, *rest):
        ou_ref, oi_ref = rest[-2], rest[-1]
        ou_ref[...] = lax.dot_general(
            x_ref[0], wu_ref[...], dn,
            preferred_element_type=jnp.float32) + bu_ref[...]
        oi_ref[...] = lax.dot_general(
            x_ref[1], wi_ref[...], dn,
            preferred_element_type=jnp.float32) + bi_ref[...]

    in_specs = [
        pl.BlockSpec((2, BM, D), lambda i: (0, i, 0)),
        pl.BlockSpec((D, D), lambda i: (0, 0)),
        pl.BlockSpec((1, D), lambda i: (0, 0)),
        pl.BlockSpec((D, D), lambda i: (0, 0)),
        pl.BlockSpec((1, D), lambda i: (0, 0)),
    ]
    inputs = [X, Wu, bu.reshape(1, D), Wi, bi.reshape(1, D)]
    aliases = {}
    if prev is not None:
        in_specs += [
            pl.BlockSpec((BM, D), lambda i, b0=blk0: (b0 + i, 0)),
            pl.BlockSpec((BM, D), lambda i, b0=blk0: (b0 + i, 0)),
        ]
        inputs += [prev[0], prev[1]]
        aliases = {5: 0, 6: 1}

    return pl.pallas_call(
        body,
        grid=(nblk,),
        in_specs=in_specs,
        out_specs=[
            pl.BlockSpec((BM, D), lambda i, b0=blk0: (b0 + i, 0)),
            pl.BlockSpec((BM, D), lambda i, b0=blk0: (b0 + i, 0)),
        ],
        out_shape=[
            jax.ShapeDtypeStruct((B, D), jnp.float32),
            jax.ShapeDtypeStruct((B, D), jnp.float32),
        ],
        input_output_aliases=aliases,
    )(*inputs)


def kernel(user_ids, item_ids, user_embedding, item_embedding,
           W_user, b_user, W_item, b_item):
    B = user_ids.shape[0]
    nunits = B // UNIT
    uids3 = user_ids.astype(jnp.int32).reshape(nunits, NW, CHUNK)
    iids3 = item_ids.astype(jnp.int32).reshape(nunits, NW, CHUNK)
    units0 = tuple(range(nunits - 1))
    units1 = (nunits - 1,)

    X0 = _sc_gather_piece(uids3, iids3, user_embedding, item_embedding,
                          units0)
    X1 = _sc_gather_piece(uids3, iids3, user_embedding, item_embedding,
                          units1)
    out0 = _tc_project_piece(X0, W_user, b_user, W_item, b_item, 0, B)
    ou, oi = _tc_project_piece(X1, W_user, b_user, W_item, b_item,
                               len(units0) * UNIT, B, prev=out0)
    return (ou, oi)


# speedup vs baseline: 1.1053x; 1.0233x over previous
import functools

import jax
import jax.numpy as jnp
from jax import lax
from jax.experimental import pallas as pl
from jax.experimental.pallas import tpu as pltpu
from jax.experimental.pallas import tpu_sc as plsc

D = 128
NC, NS = 2, 16
NW = NC * NS
CHUNK = 128
UNIT = NW * CHUNK


def _sc_gather_piece(uids3, iids3, utab, itab, units):
    kpw = len(units)
    rows = NW * kpw * CHUNK
    mesh = plsc.VectorSubcoreMesh(
        core_axis_name="c", subcore_axis_name="s",
        num_cores=NC, num_subcores=NS)

    @functools.partial(
        pl.kernel,
        out_type=jax.ShapeDtypeStruct((2, rows, D), jnp.float32),
        mesh=mesh,
        scratch_types=(
            [pltpu.VMEM((kpw, CHUNK), jnp.int32) for _ in range(2)]
            + [pltpu.VMEM((CHUNK, D), jnp.float32) for _ in range(2 * kpw)]
            + [pltpu.SemaphoreType.DMA for _ in range(6 * kpw)]
        ),
    )
    def k(uid_hbm, iid_hbm, utab_hbm, itab_hbm, out_hbm, *rest):
        idx = rest[:2]
        bufs = rest[2:2 + 2 * kpw]
        gsem = rest[2 + 2 * kpw:2 + 4 * kpw]
        wsem = rest[2 + 4 * kpw:2 + 6 * kpw]
        isem = rest[2 + 6 * kpw:]
        wid = lax.axis_index("s") * NC + lax.axis_index("c")
        stages = []
        for j, u in enumerate(units):
            stages.append(pltpu.async_copy(
                uid_hbm.at[u, wid], idx[0].at[j], isem[2 * j]))
            stages.append(pltpu.async_copy(
                iid_hbm.at[u, wid], idx[1].at[j], isem[2 * j + 1]))
        for s in stages:
            s.wait()
        gathers = []
        for p, tab in enumerate((utab_hbm, itab_hbm)):
            for j in range(kpw):
                n = p * kpw + j
                gathers.append(pltpu.async_copy(
                    tab.at[idx[p].at[j]], bufs[n], gsem[n]))
        writes = []
        for p in range(2):
            for j in range(kpw):
                n = p * kpw + j
                gathers[n].wait()
                writes.append(pltpu.async_copy(
                    bufs[n],
                    out_hbm.at[p, pl.ds((j * NW + wid) * CHUNK, CHUNK)],
                    wsem[n]))
        for w in writes:
            w.wait()

    return k(uids3, iids3, utab, itab)


def _tc_project_piece(X, Wu, bu, Wi, bi, row0, B, prev=None):
    rows = X.shape[1]
    BM = 2048
    nblk = rows // BM
    blk0 = row0 // BM
    dn = (((1,), (1,)), ((), ()))

    def body(x_ref, wu_ref, bu_ref, wi_ref, bi_ref, *rest):
        ou_ref, oi_ref = rest[-2], rest[-1]
        ou_ref[...] = lax.dot_general(
            x_ref[0], wu_ref[...], dn,
            preferred_element_type=jnp.float32) + bu_ref[...]
        oi_ref[...] = lax.dot_general(
            x_ref[1], wi_ref[...], dn,
            preferred_element_type=jnp.float32) + bi_ref[...]

    in_specs = [
        pl.BlockSpec((2, BM, D), lambda i: (0, i, 0)),
        pl.BlockSpec((D, D), lambda i: (0, 0)),
        pl.BlockSpec((1, D), lambda i: (0, 0)),
        pl.BlockSpec((D, D), lambda i: (0, 0)),
        pl.BlockSpec((1, D), lambda i: (0, 0)),
    ]
    inputs = [X, Wu, bu.reshape(1, D), Wi, bi.reshape(1, D)]
    aliases = {}
    if prev is not None:
        in_specs += [
            pl.BlockSpec(memory_space=pltpu.MemorySpace.HBM),
            pl.BlockSpec(memory_space=pltpu.MemorySpace.HBM),
        ]
        inputs += [prev[0], prev[1]]
        aliases = {5: 0, 6: 1}

    return pl.pallas_call(
        body,
        grid=(nblk,),
        in_specs=in_specs,
        out_specs=[
            pl.BlockSpec((BM, D), lambda i, b0=blk0: (b0 + i, 0)),
            pl.BlockSpec((BM, D), lambda i, b0=blk0: (b0 + i, 0)),
        ],
        out_shape=[
            jax.ShapeDtypeStruct((B, D), jnp.float32),
            jax.ShapeDtypeStruct((B, D), jnp.float32),
        ],
        input_output_aliases=aliases,
    )(*inputs)


def kernel(user_ids, item_ids, user_embedding, item_embedding,
           W_user, b_user, W_item, b_item):
    B = user_ids.shape[0]
    nunits = B // UNIT
    uids3 = user_ids.astype(jnp.int32).reshape(nunits, NW, CHUNK)
    iids3 = item_ids.astype(jnp.int32).reshape(nunits, NW, CHUNK)
    units0 = tuple(range(nunits - 1))
    units1 = (nunits - 1,)

    X0 = _sc_gather_piece(uids3, iids3, user_embedding, item_embedding,
                          units0)
    X1 = _sc_gather_piece(uids3, iids3, user_embedding, item_embedding,
                          units1)
    out0 = _tc_project_piece(X0, W_user, b_user, W_item, b_item, 0, B)
    ou, oi = _tc_project_piece(X1, W_user, b_user, W_item, b_item,
                               len(units0) * UNIT, B, prev=out0)
    return (ou, oi)
